# Initial kernel scaffold; baseline (speedup 1.0000x reference)
import jax
import jax.numpy as jnp
from jax.experimental import pallas as pl


def kernel(x, edge_index, batch, w1a, b1a, w1b, b1b, w2a, b2a, w2b, b2b, w3a, b3a, w3b, b3b, g1, be1, g2, be2, g3, be3):
    raise NotImplementedError("write your pallas kernel here")


# speedup vs baseline: 1.4970x; 1.4970x over previous
import functools

import jax
import jax.numpy as jnp
from jax import lax
from jax.experimental import pallas as pl
from jax.experimental.pallas import tpu as pltpu
from jax.experimental.pallas import tpu_sc as plsc

N = 100000
E = 1600000
H = 128
OUT = 256
NGRAPH = 512
EROWS = E // 128
ER16 = EROWS // 16
EREM = EROWS - 16 * ER16
STR_A = (ER16 + 1) * 8
STR_B = ER16 * 8
NB = 2000


def _make_sc_agg(G):
  mesh = plsc.VectorSubcoreMesh(core_axis_name="c", subcore_axis_name="s")

  @functools.partial(
      pl.kernel,
      mesh=mesh,
      compiler_params=pltpu.CompilerParams(use_tc_tiling_on_sc=False),
      out_type=[jax.ShapeDtypeStruct((N, 16), jnp.float32) for _ in range(G)],
      scratch_types=[
          pltpu.VMEM((128,), jnp.int32),
          pltpu.VMEM((128,), jnp.int32),
          pltpu.VMEM((128, 16), jnp.float32),
          pltpu.VMEM_SHARED((N, 16), jnp.float32),
          pltpu.SemaphoreType.DMA,
      ],
  )
  def agg(src_hbm, dst_hbm, zeros_hbm, *rest):
    tabs = rest[:G]
    outs = rest[G:2 * G]
    src_v, dst_v, rows_v, acc, sem = rest[2 * G:]
    cid = lax.axis_index("c")
    sid = lax.axis_index("s")
    nrows = jnp.where(sid < EREM, ER16 + 1, ER16)
    base = sid * ER16 + jnp.minimum(sid, EREM)
    sbase = sid * STR_B + jnp.minimum(sid, EREM) * 8

    def _stripe_copy(src_at, dst_at):
      @pl.when(sid < EREM)
      def _():
        pltpu.sync_copy(src_at(STR_A), dst_at(STR_A))

      @pl.when(sid >= EREM)
      def _():
        pltpu.sync_copy(src_at(STR_B), dst_at(STR_B))

    for g in range(G):
      @pl.when(cid == g // 4)
      def _(g=g):
        _stripe_copy(lambda n: zeros_hbm.at[pl.ds(sbase, n)],
                     lambda n: acc.at[pl.ds(sbase, n)])
        plsc.subcore_barrier()

        def body(r, carry):
          row = base + r
          pltpu.sync_copy(src_hbm.at[row], src_v)
          pltpu.sync_copy(dst_hbm.at[row], dst_v)
          pltpu.async_copy(tabs[g].at[src_v], rows_v, sem).wait()
          pltpu.sync_copy(rows_v, acc.at[dst_v], add=True)
          return carry

        lax.fori_loop(0, nrows, body, 0)
        plsc.subcore_barrier()
        _stripe_copy(lambda n: acc.at[pl.ds(sbase, n)],
                     lambda n: outs[g].at[pl.ds(sbase, n)])
        plsc.subcore_barrier()

  return agg


def _make_mlp(G, outd):
  nsteps = N // NB

  def body(*refs):
    hs = refs[:G]
    ags = refs[G:2 * G]
    w1, b1, w2, b2 = refs[2 * G:2 * G + 4]
    z_out, s_out, q_out = refs[2 * G + 4:]
    i = pl.program_id(0)
    acc = jnp.zeros((NB, H), jnp.float32)
    for g in range(G):
      m = hs[g][...] + ags[g][...]
      acc = acc + jnp.dot(m, w1[g], preferred_element_type=jnp.float32)
    z1 = jnp.maximum(acc + b1[...], 0.0)
    z2 = jnp.dot(z1, w2[...], preferred_element_type=jnp.float32) + b2[...]
    z_out[...] = z2
    ps = jnp.sum(z2, axis=0, keepdims=True)
    pq = jnp.sum(z2 * z2, axis=0, keepdims=True)

    @pl.when(i == 0)
    def _():
      s_out[...] = ps
      q_out[...] = pq

    @pl.when(i > 0)
    def _():
      s_out[...] += ps
      q_out[...] += pq

  node_spec = pl.BlockSpec((NB, 16), lambda i: (i, 0))
  return pl.pallas_call(
      body,
      grid=(nsteps,),
      in_specs=(
          [node_spec] * (2 * G)
          + [
              pl.BlockSpec((G, 16, H), lambda i: (0, 0, 0)),
              pl.BlockSpec((1, H), lambda i: (0, 0)),
              pl.BlockSpec((H, outd), lambda i: (0, 0)),
              pl.BlockSpec((1, outd), lambda i: (0, 0)),
          ]
      ),
      out_specs=[
          pl.BlockSpec((NB, outd), lambda i: (i, 0)),
          pl.BlockSpec((1, outd), lambda i: (0, 0)),
          pl.BlockSpec((1, outd), lambda i: (0, 0)),
      ],
      out_shape=[
          jax.ShapeDtypeStruct((N, outd), jnp.float32),
          jax.ShapeDtypeStruct((1, outd), jnp.float32),
          jax.ShapeDtypeStruct((1, outd), jnp.float32),
      ],
  )


def _make_bn(G, d, relu):
  nsteps = N // NB

  def body(z_ref, s_ref, q_ref, ga_ref, be_ref, *outs):
    mean = s_ref[...] / N
    var = q_ref[...] / N - mean * mean
    inv = ga_ref[...] * lax.rsqrt(var + 1e-5)
    h = (z_ref[...] - mean) * inv + be_ref[...]
    if relu:
      h = jnp.maximum(h, 0.0)
    for g in range(G):
      outs[g][...] = h[:, g * 16:(g + 1) * 16]

  return pl.pallas_call(
      body,
      grid=(nsteps,),
      in_specs=[
          pl.BlockSpec((NB, d), lambda i: (i, 0)),
          pl.BlockSpec((1, d), lambda i: (0, 0)),
          pl.BlockSpec((1, d), lambda i: (0, 0)),
          pl.BlockSpec((1, d), lambda i: (0, 0)),
          pl.BlockSpec((1, d), lambda i: (0, 0)),
      ],
      out_specs=[pl.BlockSpec((NB, 16), lambda i: (i, 0))] * G,
      out_shape=[jax.ShapeDtypeStruct((N, 16), jnp.float32)] * G,
  )


def _make_pool():
  nsteps = N // NB

  def body(z_ref, s_ref, q_ref, ga_ref, be_ref, b_ref, o_ref, sums, cnts):
    i = pl.program_id(0)
    mean = s_ref[...] / N
    var = q_ref[...] / N - mean * mean
    inv = ga_ref[...] * lax.rsqrt(var + 1e-5)
    h = (z_ref[...] - mean) * inv + be_ref[...]
    oh = (lax.broadcasted_iota(jnp.int32, (NB, NGRAPH), 1)
          == b_ref[...]).astype(jnp.float32)
    dn = (((0,), (0,)), ((), ()))
    ps = lax.dot_general(oh, h, dn, preferred_element_type=jnp.float32)
    pc = lax.dot_general(oh, jnp.ones((NB, 1), jnp.float32), dn,
                         preferred_element_type=jnp.float32)

    @pl.when(i == 0)
    def _():
      sums[...] = ps
      cnts[...] = pc

    @pl.when(i > 0)
    def _():
      sums[...] += ps
      cnts[...] += pc

    o_ref[...] = sums[...] / jnp.maximum(cnts[...], 1.0)

  return pl.pallas_call(
      body,
      grid=(nsteps,),
      in_specs=[
          pl.BlockSpec((NB, OUT), lambda i: (i, 0)),
          pl.BlockSpec((1, OUT), lambda i: (0, 0)),
          pl.BlockSpec((1, OUT), lambda i: (0, 0)),
          pl.BlockSpec((1, OUT), lambda i: (0, 0)),
          pl.BlockSpec((1, OUT), lambda i: (0, 0)),
          pl.BlockSpec((NB, 1), lambda i: (i, 0)),
      ],
      out_specs=pl.BlockSpec((NGRAPH, OUT), lambda i: (0, 0)),
      out_shape=jax.ShapeDtypeStruct((NGRAPH, OUT), jnp.float32),
      scratch_shapes=[
          pltpu.VMEM((NGRAPH, OUT), jnp.float32),
          pltpu.VMEM((NGRAPH, 1), jnp.float32),
      ],
  )


_sc_agg1 = _make_sc_agg(1)
_sc_agg8 = _make_sc_agg(8)
_mlp1 = _make_mlp(1, H)
_mlp8h = _make_mlp(8, H)
_mlp8o = _make_mlp(8, OUT)
_bn_relu1 = _make_bn(8, H, True)
_pool = _make_pool()


def kernel(x, edge_index, batch,
           w1a, b1a, w1b, b1b,
           w2a, b2a, w2b, b2b,
           w3a, b3a, w3b, b3b,
           g1, be1, g2, be2, g3, be3):
  src2d = edge_index[0].reshape(EROWS, 128)
  dst2d = edge_index[1].reshape(EROWS, 128)
  zeros = jnp.zeros((N, 16), jnp.float32)
  xp = jnp.pad(x, ((0, 0), (0, 7)))
  w1 = jnp.pad(w1a, ((0, 7), (0, 0))).reshape(1, 16, H)

  a1 = _sc_agg1(src2d, dst2d, zeros, xp)
  a1 = a1 if isinstance(a1, (list, tuple)) else [a1]
  z1, s1, q1 = _mlp1(xp, a1[0], w1, b1a.reshape(1, H), w1b,
                     b1b.reshape(1, H))
  h1 = _bn_relu1(z1, s1, q1, g1.reshape(1, H), be1.reshape(1, H))

  a2 = _sc_agg8(src2d, dst2d, zeros, *h1)
  z2, s2, q2 = _mlp8h(*h1, *a2, w2a.reshape(8, 16, H),
                      b2a.reshape(1, H), w2b, b2b.reshape(1, H))
  h2 = _bn_relu1(z2, s2, q2, g2.reshape(1, H), be2.reshape(1, H))

  a3 = _sc_agg8(src2d, dst2d, zeros, *h2)
  z3, s3, q3 = _mlp8o(*h2, *a3, w3a.reshape(8, 16, H),
                      b3a.reshape(1, H), w3b, b3b.reshape(1, OUT))
  return _pool(z3, s3, q3, g3.reshape(1, OUT), be3.reshape(1, OUT),
               batch.reshape(N, 1))
